# Initial kernel scaffold; baseline (speedup 1.0000x reference)
import jax
import jax.numpy as jnp
from jax.experimental import pallas as pl


def kernel(feat, edge_index, weight1):
    raise NotImplementedError("write your pallas kernel here")


# speedup vs baseline: 4.9674x; 4.9674x over previous
import functools

import jax
import jax.numpy as jnp
from jax import lax
from jax.experimental import pallas as pl
from jax.experimental.pallas import tpu as pltpu
from jax.experimental.pallas import tpu_sc as plsc

N_NODES_C = 10000
N_EDGES_C = 320000
D_C = 128

NC = 2
NS = 16
NW = NC * NS
LANES = 16

EPT = N_EDGES_C // NW
CH = 128
NCHUNK = -(-EPT // CH)
PAD_EPT = NCHUNK * CH
DUMMY = N_NODES_C
ACC_PER_TILE = 640
ACC_ROWS = NS * ACC_PER_TILE


def _sc_body(feat_hbm, src_hbm, dst_hbm, out_hbm,
             src2d, dst2d, rows, acc):
    c = lax.axis_index("c")
    s = lax.axis_index("s")
    w = c * NS + s

    @pl.loop(0, CH)
    def _(i):
        @pl.loop(0, D_C // LANES)
        def _(k):
            rows[i, pl.ds(k * LANES, LANES)] = jnp.zeros((LANES,), jnp.float32)

    @pl.loop(0, ACC_PER_TILE // CH)
    def _(j):
        pltpu.sync_copy(rows, acc.at[pl.ds(s * ACC_PER_TILE + j * CH, CH)])

    pltpu.sync_copy(src_hbm.at[w], src2d)
    pltpu.sync_copy(dst_hbm.at[w], dst2d)

    @pl.loop(0, NCHUNK)
    def _(r):
        @pl.loop(0, CH // LANES)
        def _(k):
            sv = src2d[r, pl.ds(k * LANES, LANES)]
            dv = dst2d[r, pl.ds(k * LANES, LANES)]
            dst2d[r, pl.ds(k * LANES, LANES)] = jnp.where(
                sv == dv, jnp.full((LANES,), DUMMY, jnp.int32), dv)

    plsc.subcore_barrier()

    @pl.loop(0, NCHUNK)
    def _(cc):
        pltpu.sync_copy(feat_hbm.at[src2d.at[cc]], rows)
        pltpu.sync_copy(rows, acc.at[dst2d.at[cc]], add=True)

    plsc.subcore_barrier()

    pltpu.sync_copy(acc.at[pl.ds(s * ACC_PER_TILE, ACC_PER_TILE)],
                    out_hbm.at[c].at[pl.ds(s * ACC_PER_TILE, ACC_PER_TILE)])


@jax.jit
def _sc_scatter(feat, srcp, dstp):
    mesh = plsc.VectorSubcoreMesh(core_axis_name="c", subcore_axis_name="s")
    k = pl.kernel(
        _sc_body,
        out_type=jax.ShapeDtypeStruct((NC, ACC_ROWS, D_C), jnp.float32),
        mesh=mesh,
        scratch_types=[
            pltpu.VMEM((NCHUNK, CH), jnp.int32),
            pltpu.VMEM((NCHUNK, CH), jnp.int32),
            pltpu.VMEM((CH, D_C), jnp.float32),
            pltpu.VMEM_SHARED((ACC_ROWS, D_C), jnp.float32),
        ],
    )
    return k(feat, srcp, dstp)


def _finish_body(p0_ref, p1_ref, f_ref, w_ref, o_ref):
    x = p0_ref[...] + p1_ref[...] + f_ref[...]
    y = lax.dot_general(x, w_ref[...], (((1,), (0,)), ((), ())),
                        preferred_element_type=jnp.float32,
                        precision=lax.Precision.HIGHEST)
    o_ref[...] = jnp.maximum(y, 0.0)


BR = 400


@jax.jit
def _tc_finish(p0, p1, feat, weight1):
    return pl.pallas_call(
        _finish_body,
        grid=(N_NODES_C // BR,),
        in_specs=[
            pl.BlockSpec((BR, D_C), lambda i: (i, 0)),
            pl.BlockSpec((BR, D_C), lambda i: (i, 0)),
            pl.BlockSpec((BR, D_C), lambda i: (i, 0)),
            pl.BlockSpec((D_C, D_C), lambda i: (0, 0)),
        ],
        out_specs=pl.BlockSpec((BR, D_C), lambda i: (i, 0)),
        out_shape=jax.ShapeDtypeStruct((N_NODES_C, D_C), jnp.float32),
    )(p0, p1, feat, weight1)


def kernel(feat, edge_index, weight1):
    src = edge_index[0].astype(jnp.int32)
    dst = edge_index[1].astype(jnp.int32)
    npad = NW * PAD_EPT - N_EDGES_C
    srcp = jnp.concatenate(
        [src, jnp.zeros((npad,), jnp.int32)]).reshape(NW, NCHUNK, CH)
    dstp = jnp.concatenate(
        [dst, jnp.full((npad,), DUMMY, jnp.int32)]).reshape(NW, NCHUNK, CH)
    partials = _sc_scatter(feat, srcp, dstp)
    return _tc_finish(partials[0, :N_NODES_C], partials[1, :N_NODES_C],
                      feat, weight1)


# speedup vs baseline: 6.8901x; 1.3871x over previous
import functools

import jax
import jax.numpy as jnp
from jax import lax
from jax.experimental import pallas as pl
from jax.experimental.pallas import tpu as pltpu
from jax.experimental.pallas import tpu_sc as plsc

N_NODES_C = 10000
N_EDGES_C = 320000
D_C = 128

NC = 2
NS = 16
NW = NC * NS
LANES = 16

CH = 112
NCHUNK = 90
DUMMY = N_NODES_C
ACC_PER_TILE = 640
ACC_ROWS = NS * ACC_PER_TILE


def _sc_body(feat_hbm, src_hbm, dst_hbm, out_hbm, *refs):
    sidx = refs[0:3]
    didx = refs[3:6]
    rows = refs[6:9]
    acc = refs[9]
    isems = refs[10:13]
    gsems = refs[13:16]
    ssems = refs[16:19]

    c = lax.axis_index("c")
    s = lax.axis_index("s")
    w = c * NS + s
    cbase = w * NCHUNK

    def idx_start(b, k):
        pltpu.async_copy(src_hbm.at[cbase + k], sidx[b], isems[b])
        pltpu.async_copy(dst_hbm.at[cbase + k], didx[b], isems[b])

    def idx_wait(b):
        pltpu.make_async_copy(src_hbm.at[0], sidx[b], isems[b]).wait()
        pltpu.make_async_copy(dst_hbm.at[0], didx[b], isems[b]).wait()

    def mask(b):
        for j in range(CH // LANES):
            sl = pl.ds(j * LANES, LANES)
            sv = sidx[b][0, sl]
            dv = didx[b][0, sl]
            didx[b][0, sl] = jnp.where(
                sv == dv, jnp.full((LANES,), DUMMY, jnp.int32), dv)

    def g_start(b):
        pltpu.async_copy(feat_hbm.at[sidx[b].at[0]], rows[b], gsems[b])

    def g_wait(b):
        pltpu.make_async_copy(
            feat_hbm.at[sidx[b].at[0]], rows[b], gsems[b]).wait()

    def s_start(b):
        pltpu.async_copy(rows[b], acc.at[didx[b].at[0]], ssems[b], add=True)

    def s_wait(b):
        pltpu.make_async_copy(
            rows[b], acc.at[didx[b].at[0]], ssems[b]).wait()

    @pl.loop(0, CH)
    def _(i):
        @pl.loop(0, D_C // LANES)
        def _(k):
            rows[0][i, pl.ds(k * LANES, LANES)] = jnp.zeros(
                (LANES,), jnp.float32)

    abase = s * ACC_PER_TILE

    @pl.loop(0, 5)
    def _(j):
        pltpu.sync_copy(rows[0], acc.at[pl.ds(abase + j * CH, CH)])
    pltpu.sync_copy(rows[0].at[pl.ds(0, ACC_PER_TILE - 5 * CH)],
                    acc.at[pl.ds(abase + 5 * CH, ACC_PER_TILE - 5 * CH)])

    plsc.subcore_barrier()

    def do_iter(k, b, f_next=True, f_sw2=True, f_idx=True):
        b1, b2 = (b + 1) % 3, (b + 2) % 3
        if f_next:
            idx_wait(b1)
            mask(b1)
            g_start(b1)
        g_wait(b)
        s_start(b)
        if f_sw2:
            s_wait(b2)
        if f_idx:
            idx_start(b2, k + 2)

    idx_start(0, 0)
    idx_start(1, 1)
    idx_wait(0)
    mask(0)
    g_start(0)

    do_iter(0, 0, f_sw2=False)
    do_iter(1, 1)
    do_iter(2, 2)

    @pl.loop(3, NCHUNK - 3, step=3)
    def _(k):
        do_iter(k, 0)
        do_iter(k + 1, 1)
        do_iter(k + 2, 2)

    do_iter(NCHUNK - 3, 0)
    do_iter(NCHUNK - 2, 1, f_idx=False)
    do_iter(NCHUNK - 1, 2, f_next=False, f_idx=False)
    s_wait(2)

    plsc.subcore_barrier()

    pltpu.sync_copy(acc.at[pl.ds(abase, ACC_PER_TILE)],
                    out_hbm.at[c].at[pl.ds(abase, ACC_PER_TILE)])


@jax.jit
def _sc_scatter(feat, srcp, dstp):
    mesh = plsc.VectorSubcoreMesh(core_axis_name="c", subcore_axis_name="s")
    k = pl.kernel(
        _sc_body,
        out_type=jax.ShapeDtypeStruct((NC, ACC_ROWS, D_C), jnp.float32),
        mesh=mesh,
        scratch_types=(
            [pltpu.VMEM((1, CH), jnp.int32) for _ in range(6)]
            + [pltpu.VMEM((CH, D_C), jnp.float32) for _ in range(3)]
            + [pltpu.VMEM_SHARED((ACC_ROWS, D_C), jnp.float32)]
            + [pltpu.SemaphoreType.DMA for _ in range(9)]
        ),
    )
    return k(feat, srcp, dstp)


def _finish_body(p0_ref, p1_ref, f_ref, w_ref, o_ref):
    x = p0_ref[...] + p1_ref[...] + f_ref[...]
    y = lax.dot_general(x, w_ref[...], (((1,), (0,)), ((), ())),
                        preferred_element_type=jnp.float32,
                        precision=lax.Precision.HIGHEST)
    o_ref[...] = jnp.maximum(y, 0.0)


BR = 400


@jax.jit
def _tc_finish(p0, p1, feat, weight1):
    return pl.pallas_call(
        _finish_body,
        grid=(N_NODES_C // BR,),
        in_specs=[
            pl.BlockSpec((BR, D_C), lambda i: (i, 0)),
            pl.BlockSpec((BR, D_C), lambda i: (i, 0)),
            pl.BlockSpec((BR, D_C), lambda i: (i, 0)),
            pl.BlockSpec((D_C, D_C), lambda i: (0, 0)),
        ],
        out_specs=pl.BlockSpec((BR, D_C), lambda i: (i, 0)),
        out_shape=jax.ShapeDtypeStruct((N_NODES_C, D_C), jnp.float32),
    )(p0, p1, feat, weight1)


def kernel(feat, edge_index, weight1):
    src = edge_index[0].astype(jnp.int32)
    dst = edge_index[1].astype(jnp.int32)
    npad = NW * NCHUNK * CH - N_EDGES_C
    srcp = jnp.concatenate(
        [src, jnp.zeros((npad,), jnp.int32)]).reshape(NW * NCHUNK, 1, CH)
    dstp = jnp.concatenate(
        [dst, jnp.full((npad,), DUMMY, jnp.int32)]).reshape(NW * NCHUNK, 1, CH)
    partials = _sc_scatter(feat, srcp, dstp)
    return _tc_finish(partials[0, :N_NODES_C], partials[1, :N_NODES_C],
                      feat, weight1)


# speedup vs baseline: 10.2100x; 1.4818x over previous
import functools

import jax
import jax.numpy as jnp
from jax import lax
from jax.experimental import pallas as pl
from jax.experimental.pallas import tpu as pltpu
from jax.experimental.pallas import tpu_sc as plsc

N_NODES_C = 10000
N_EDGES_C = 320000
D_C = 128

NC = 2
NS = 16
LANES = 16

CH = 80
NCH0 = 171
NCH1 = 79
DUMMY = N_NODES_C
ACC_PER_TILE = 640
ACC_ROWS = NS * ACC_PER_TILE


def _sc_body(feat_hbm, eflat_hbm, out_hbm, *refs):
    sidx = refs[0:3]
    dbuf = refs[3:6]
    didx = refs[6:9]
    rows = refs[9:12]
    acc = refs[12]
    isems = refs[13:16]
    gsems = refs[16:19]
    ssems = refs[19:22]

    c = lax.axis_index("c")
    s = lax.axis_index("s")

    def idx_start(b, eoff):
        pltpu.async_copy(eflat_hbm.at[pl.ds(eoff, CH)], sidx[b], isems[b])
        pltpu.async_copy(eflat_hbm.at[pl.ds(N_EDGES_C + eoff, CH)],
                         dbuf[b], isems[b])

    def idx_wait(b):
        pltpu.make_async_copy(
            eflat_hbm.at[pl.ds(0, CH)], sidx[b], isems[b]).wait()
        pltpu.make_async_copy(
            eflat_hbm.at[pl.ds(0, CH)], dbuf[b], isems[b]).wait()

    def mask(b):
        for j in range(CH // LANES):
            sl = pl.ds(j * LANES, LANES)
            sv = sidx[b][sl]
            dv = dbuf[b][sl]
            didx[b][0, sl] = jnp.where(
                sv == dv, jnp.full((LANES,), DUMMY, jnp.int32), dv)

    def g_start(b):
        pltpu.async_copy(feat_hbm.at[sidx[b]], rows[b], gsems[b])

    def g_wait(b):
        pltpu.make_async_copy(feat_hbm.at[sidx[b]], rows[b], gsems[b]).wait()

    def s_start(b):
        pltpu.async_copy(rows[b], acc.at[didx[b].at[0]], ssems[b], add=True)

    def s_wait(b):
        pltpu.make_async_copy(
            rows[b], acc.at[didx[b].at[0]], ssems[b]).wait()

    @pl.loop(0, CH)
    def _(i):
        @pl.loop(0, D_C // LANES)
        def _(k):
            rows[0][i, pl.ds(k * LANES, LANES)] = jnp.zeros(
                (LANES,), jnp.float32)

    abase = s * ACC_PER_TILE

    @pl.loop(0, ACC_PER_TILE // CH)
    def _(j):
        pltpu.sync_copy(rows[0], acc.at[pl.ds(abase + j * CH, CH)])

    plsc.subcore_barrier()

    def do_iter(k, b, ebase, n, k_traced=None):
        kk = k if k_traced is None else k_traced
        b1, b2 = (b + 1) % 3, (b + 2) % 3
        if k + 1 < n:
            idx_wait(b1)
            mask(b1)
            g_start(b1)
        g_wait(b)
        s_start(b)
        if k >= 1:
            s_wait(b2)
        if k + 2 < n:
            idx_start(b2, ebase + (kk + 2) * CH)

    def pipeline(ebase, n):
        idx_start(0, ebase)
        idx_start(1, ebase + CH)
        idx_wait(0)
        mask(0)
        g_start(0)

        nmain = ((n - 6) // 3) * 3

        for k in range(0, 3):
            do_iter(k, k % 3, ebase, n)

        @pl.loop(3, 3 + nmain, step=3)
        def _(k):
            do_iter(4, 0, ebase, n, k_traced=k)
            do_iter(4, 1, ebase, n, k_traced=k + 1)
            do_iter(4, 2, ebase, n, k_traced=k + 2)

        for k in range(3 + nmain, n):
            do_iter(k, k % 3, ebase, n)
        s_wait((n - 1) % 3)

    @pl.when(c == 0)
    def _():
        pipeline(s * (CH * NCH0), NCH0)

    @pl.when(c == 1)
    def _():
        pipeline(NS * CH * NCH0 + s * (CH * NCH1), NCH1)

    plsc.subcore_barrier()

    pltpu.sync_copy(acc.at[pl.ds(abase, ACC_PER_TILE)],
                    out_hbm.at[c].at[pl.ds(abase, ACC_PER_TILE)])


@jax.jit
def _sc_scatter(feat, eflat):
    mesh = plsc.VectorSubcoreMesh(core_axis_name="c", subcore_axis_name="s")
    k = pl.kernel(
        _sc_body,
        out_type=jax.ShapeDtypeStruct((NC, ACC_ROWS, D_C), jnp.float32),
        mesh=mesh,
        scratch_types=(
            [pltpu.VMEM((CH,), jnp.int32) for _ in range(6)]
            + [pltpu.VMEM((1, CH), jnp.int32) for _ in range(3)]
            + [pltpu.VMEM((CH, D_C), jnp.float32) for _ in range(3)]
            + [pltpu.VMEM_SHARED((ACC_ROWS, D_C), jnp.float32)]
            + [pltpu.SemaphoreType.DMA for _ in range(9)]
        ),
    )
    return k(feat, eflat)


def _finish_body(p0_ref, p1_ref, f_ref, w_ref, o_ref):
    x = p0_ref[0] + p1_ref[0] + f_ref[...]
    y = lax.dot_general(x, w_ref[...], (((1,), (0,)), ((), ())),
                        preferred_element_type=jnp.float32,
                        precision=lax.Precision.HIGHEST)
    o_ref[...] = jnp.maximum(y, 0.0)


BR = 400


@jax.jit
def _tc_finish(partials, feat, weight1):
    return pl.pallas_call(
        _finish_body,
        grid=(N_NODES_C // BR,),
        in_specs=[
            pl.BlockSpec((1, BR, D_C), lambda i: (0, i, 0)),
            pl.BlockSpec((1, BR, D_C), lambda i: (1, i, 0)),
            pl.BlockSpec((BR, D_C), lambda i: (i, 0)),
            pl.BlockSpec((D_C, D_C), lambda i: (0, 0)),
        ],
        out_specs=pl.BlockSpec((BR, D_C), lambda i: (i, 0)),
        out_shape=jax.ShapeDtypeStruct((N_NODES_C, D_C), jnp.float32),
    )(partials, partials, feat, weight1)


def kernel(feat, edge_index, weight1):
    eflat = edge_index.astype(jnp.int32).reshape(2 * N_EDGES_C)
    partials = _sc_scatter(feat, eflat)
    return _tc_finish(partials, feat, weight1)


# speedup vs baseline: 12.3801x; 1.2126x over previous
import functools

import jax
import jax.numpy as jnp
from jax import lax
from jax.experimental import pallas as pl
from jax.experimental.pallas import tpu as pltpu
from jax.experimental.pallas import tpu_sc as plsc

N_NODES_C = 10000
N_EDGES_C = 320000
D_C = 128

NC = 2
NS = 16
LANES = 16

CH = 80
NCH0 = 130
NCH1 = 120
DUMMY = N_NODES_C
ACC_PER_TILE = 640
ACC_ROWS = NS * ACC_PER_TILE


def _sc_body(feat_hbm, eflat_hbm, out_hbm, *refs):
    sidx = refs[0:3]
    dbuf = refs[3:6]
    didx = refs[6:9]
    rows = refs[9:12]
    acc = refs[12]
    isems = refs[13:16]
    gsems = refs[16:19]
    ssems = refs[19:22]

    c = lax.axis_index("c")
    s = lax.axis_index("s")

    def idx_start(b, eoff):
        pltpu.async_copy(eflat_hbm.at[pl.ds(eoff, CH)], sidx[b], isems[b])
        pltpu.async_copy(eflat_hbm.at[pl.ds(N_EDGES_C + eoff, CH)],
                         dbuf[b], isems[b])

    def idx_wait(b):
        pltpu.make_async_copy(
            eflat_hbm.at[pl.ds(0, CH)], sidx[b], isems[b]).wait()
        pltpu.make_async_copy(
            eflat_hbm.at[pl.ds(0, CH)], dbuf[b], isems[b]).wait()

    def mask(b):
        for j in range(CH // LANES):
            sl = pl.ds(j * LANES, LANES)
            sv = sidx[b][sl]
            dv = dbuf[b][sl]
            didx[b][0, sl] = jnp.where(
                sv == dv, jnp.full((LANES,), DUMMY, jnp.int32), dv)

    def g_start(b):
        pltpu.async_copy(feat_hbm.at[sidx[b]], rows[b], gsems[b])

    def g_wait(b):
        pltpu.make_async_copy(feat_hbm.at[sidx[b]], rows[b], gsems[b]).wait()

    def s_start(b):
        pltpu.async_copy(rows[b], acc.at[didx[b].at[0]], ssems[b], add=True)

    def s_wait(b):
        pltpu.make_async_copy(
            rows[b], acc.at[didx[b].at[0]], ssems[b]).wait()

    @pl.loop(0, CH)
    def _(i):
        @pl.loop(0, D_C // LANES)
        def _(k):
            rows[0][i, pl.ds(k * LANES, LANES)] = jnp.zeros(
                (LANES,), jnp.float32)

    abase = s * ACC_PER_TILE

    @pl.loop(0, ACC_PER_TILE // CH)
    def _(j):
        pltpu.sync_copy(rows[0], acc.at[pl.ds(abase + j * CH, CH)])

    plsc.subcore_barrier()

    def do_iter(k, b, ebase, n, k_traced=None):
        kk = k if k_traced is None else k_traced
        b1, b2 = (b + 1) % 3, (b + 2) % 3
        if k + 1 < n:
            idx_wait(b1)
            mask(b1)
            g_start(b1)
        g_wait(b)
        s_start(b)
        if k >= 1:
            s_wait(b2)
        if k + 2 < n:
            idx_start(b2, ebase + (kk + 2) * CH)

    def pipeline(ebase, n):
        idx_start(0, ebase)
        idx_start(1, ebase + CH)
        idx_wait(0)
        mask(0)
        g_start(0)

        nmain = ((n - 6) // 3) * 3

        for k in range(0, 3):
            do_iter(k, k % 3, ebase, n)

        @pl.loop(3, 3 + nmain, step=3)
        def _(k):
            do_iter(4, 0, ebase, n, k_traced=k)
            do_iter(4, 1, ebase, n, k_traced=k + 1)
            do_iter(4, 2, ebase, n, k_traced=k + 2)

        for k in range(3 + nmain, n):
            do_iter(k, k % 3, ebase, n)
        s_wait((n - 1) % 3)

    @pl.when(c == 0)
    def _():
        pipeline(s * (CH * NCH0), NCH0)

    @pl.when(c == 1)
    def _():
        pipeline(NS * CH * NCH0 + s * (CH * NCH1), NCH1)

    plsc.subcore_barrier()

    pltpu.sync_copy(acc.at[pl.ds(abase, ACC_PER_TILE)],
                    out_hbm.at[c].at[pl.ds(abase, ACC_PER_TILE)])


@jax.jit
def _sc_scatter(feat, eflat):
    mesh = plsc.VectorSubcoreMesh(core_axis_name="c", subcore_axis_name="s")
    k = pl.kernel(
        _sc_body,
        out_type=jax.ShapeDtypeStruct((NC, ACC_ROWS, D_C), jnp.float32),
        mesh=mesh,
        scratch_types=(
            [pltpu.VMEM((CH,), jnp.int32) for _ in range(6)]
            + [pltpu.VMEM((1, CH), jnp.int32) for _ in range(3)]
            + [pltpu.VMEM((CH, D_C), jnp.float32) for _ in range(3)]
            + [pltpu.VMEM_SHARED((ACC_ROWS, D_C), jnp.float32)]
            + [pltpu.SemaphoreType.DMA for _ in range(9)]
        ),
    )
    return k(feat, eflat)


def _finish_body(p0_ref, p1_ref, f_ref, w_ref, o_ref):
    x = p0_ref[0] + p1_ref[0] + f_ref[...]
    y = lax.dot_general(x, w_ref[...], (((1,), (0,)), ((), ())),
                        preferred_element_type=jnp.float32,
                        precision=lax.Precision.DEFAULT)
    o_ref[...] = jnp.maximum(y, 0.0)


BR = 400


@jax.jit
def _tc_finish(partials, feat, weight1):
    return pl.pallas_call(
        _finish_body,
        grid=(N_NODES_C // BR,),
        in_specs=[
            pl.BlockSpec((1, BR, D_C), lambda i: (0, i, 0)),
            pl.BlockSpec((1, BR, D_C), lambda i: (1, i, 0)),
            pl.BlockSpec((BR, D_C), lambda i: (i, 0)),
            pl.BlockSpec((D_C, D_C), lambda i: (0, 0)),
        ],
        out_specs=pl.BlockSpec((BR, D_C), lambda i: (i, 0)),
        out_shape=jax.ShapeDtypeStruct((N_NODES_C, D_C), jnp.float32),
    )(partials, partials, feat, weight1)


def kernel(feat, edge_index, weight1):
    eflat = edge_index.astype(jnp.int32).reshape(2 * N_EDGES_C)
    partials = _sc_scatter(feat, eflat)
    return _tc_finish(partials, feat, weight1)


# speedup vs baseline: 15.2166x; 1.2291x over previous
import functools

import jax
import jax.numpy as jnp
from jax import lax
from jax.experimental import pallas as pl
from jax.experimental.pallas import tpu as pltpu
from jax.experimental.pallas import tpu_sc as plsc

N_NODES_C = 10000
N_EDGES_C = 320000
D_C = 128

NC = 2
NS = 16
LANES = 16

CH = 128
NCHT = 78
NXTRA = 4
DUMMY = N_NODES_C
ACC_PER_TILE = 632
ACC_ROWS = NS * ACC_PER_TILE


def _sc_body(feat_hbm, ei_hbm, out_hbm, *refs):
    idx = refs[0:3]
    rows = refs[3:6]
    acc = refs[6]
    isems = refs[7:10]
    gsems = refs[10:13]
    ssems = refs[13:16]

    c = lax.axis_index("c")
    s = lax.axis_index("s")
    w = c * NS + s

    def idx_start(b, eoff):
        pltpu.async_copy(
            ei_hbm.at[pl.ds(0, 2), pl.ds(eoff, CH)], idx[b], isems[b])

    def idx_wait(b):
        pltpu.make_async_copy(
            ei_hbm.at[pl.ds(0, 2), pl.ds(0, CH)], idx[b], isems[b]).wait()

    def mask(b):
        for j in range(CH // LANES):
            sl = pl.ds(j * LANES, LANES)
            sv = idx[b][0, sl]
            dv = idx[b][1, sl]
            idx[b][1, sl] = jnp.where(
                sv == dv, jnp.full((LANES,), DUMMY, jnp.int32), dv)

    def g_start(b):
        pltpu.async_copy(feat_hbm.at[idx[b].at[0]], rows[b], gsems[b])

    def g_wait(b):
        pltpu.make_async_copy(feat_hbm.at[idx[b].at[0]], rows[b],
                              gsems[b]).wait()

    def s_start(b):
        pltpu.async_copy(rows[b], acc.at[idx[b].at[1]], ssems[b], add=True)

    def s_wait(b):
        pltpu.make_async_copy(rows[b], acc.at[idx[b].at[1]], ssems[b]).wait()

    @pl.loop(0, CH)
    def _(i):
        @pl.loop(0, D_C // LANES)
        def _(k):
            rows[0][i, pl.ds(k * LANES, LANES)] = jnp.zeros(
                (LANES,), jnp.float32)

    abase = s * ACC_PER_TILE

    @pl.loop(0, 4)
    def _(j):
        pltpu.sync_copy(rows[0], acc.at[pl.ds(abase + j * CH, CH)])
    pltpu.sync_copy(rows[0].at[pl.ds(0, ACC_PER_TILE - 4 * CH)],
                    acc.at[pl.ds(abase + 4 * CH, ACC_PER_TILE - 4 * CH)])

    plsc.subcore_barrier()

    ebase = w * (NCHT * CH)
    xoff = (NS * NC * NCHT + 2 * c + s) * CH

    def eoff_of(kk, static_k, n):
        if static_k == n - 1 and n == NCHT + 1:
            return xoff
        return ebase + kk * CH

    def do_iter(k, b, n, k_traced=None):
        kk = k if k_traced is None else k_traced
        b1, b2 = (b + 1) % 3, (b + 2) % 3
        if k + 1 < n:
            idx_wait(b1)
            mask(b1)
            g_start(b1)
        g_wait(b)
        s_start(b)
        if k >= 1:
            s_wait(b2)
        if k + 2 < n:
            idx_start(b2, eoff_of(kk + 2, k + 2, n))

    def pipeline(n):
        idx_start(0, eoff_of(0, 0, n))
        idx_start(1, eoff_of(1, 1, n))
        idx_wait(0)
        mask(0)
        g_start(0)

        nmain = ((n - 6) // 3) * 3

        for k in range(0, 3):
            do_iter(k, k % 3, n)

        @pl.loop(3, 3 + nmain, step=3)
        def _(k):
            do_iter(4, 0, n, k_traced=k)
            do_iter(4, 1, n, k_traced=k + 1)
            do_iter(4, 2, n, k_traced=k + 2)

        for k in range(3 + nmain, n):
            do_iter(k, k % 3, n)
        s_wait((n - 1) % 3)

    @pl.when(s < 2)
    def _():
        pipeline(NCHT + 1)

    @pl.when(s >= 2)
    def _():
        pipeline(NCHT)

    plsc.subcore_barrier()

    pltpu.sync_copy(acc.at[pl.ds(abase, ACC_PER_TILE)],
                    out_hbm.at[c].at[pl.ds(abase, ACC_PER_TILE)])


@jax.jit
def _sc_scatter(feat, edge_index):
    mesh = plsc.VectorSubcoreMesh(core_axis_name="c", subcore_axis_name="s")
    k = pl.kernel(
        _sc_body,
        out_type=jax.ShapeDtypeStruct((NC, ACC_ROWS, D_C), jnp.float32),
        mesh=mesh,
        scratch_types=(
            [pltpu.VMEM((2, CH), jnp.int32) for _ in range(3)]
            + [pltpu.VMEM((CH, D_C), jnp.float32) for _ in range(3)]
            + [pltpu.VMEM_SHARED((ACC_ROWS, D_C), jnp.float32)]
            + [pltpu.SemaphoreType.DMA for _ in range(9)]
        ),
    )
    return k(feat, edge_index)


def _finish_body(p0_ref, p1_ref, f_ref, w_ref, o_ref):
    x = p0_ref[0] + p1_ref[0] + f_ref[...]
    y = lax.dot_general(x, w_ref[...], (((1,), (0,)), ((), ())),
                        preferred_element_type=jnp.float32,
                        precision=lax.Precision.DEFAULT)
    o_ref[...] = jnp.maximum(y, 0.0)


BR = 1000


@jax.jit
def _tc_finish(partials, feat, weight1):
    return pl.pallas_call(
        _finish_body,
        grid=(N_NODES_C // BR,),
        in_specs=[
            pl.BlockSpec((1, BR, D_C), lambda i: (0, i, 0)),
            pl.BlockSpec((1, BR, D_C), lambda i: (1, i, 0)),
            pl.BlockSpec((BR, D_C), lambda i: (i, 0)),
            pl.BlockSpec((D_C, D_C), lambda i: (0, 0)),
        ],
        out_specs=pl.BlockSpec((BR, D_C), lambda i: (i, 0)),
        out_shape=jax.ShapeDtypeStruct((N_NODES_C, D_C), jnp.float32),
    )(partials, partials, feat, weight1)


def kernel(feat, edge_index, weight1):
    ei = edge_index.astype(jnp.int32)
    partials = _sc_scatter(feat, ei)
    return _tc_finish(partials, feat, weight1)


# speedup vs baseline: 15.5486x; 1.0218x over previous
import functools

import jax
import jax.numpy as jnp
from jax import lax
from jax.experimental import pallas as pl
from jax.experimental.pallas import tpu as pltpu
from jax.experimental.pallas import tpu_sc as plsc

N_NODES_C = 10000
N_EDGES_C = 320000
D_C = 128

NC = 2
NS = 16
LANES = 16

CH = 128
NCHT = 78
NXTRA = 4
DUMMY = N_NODES_C
ACC_PER_TILE = 632
ACC_ROWS = NS * ACC_PER_TILE


def _sc_body(feat_hbm, ei_hbm, out_hbm, *refs):
    idx = refs[0:3]
    rows = refs[3:6]
    acc = refs[6]
    isems = refs[7:10]
    gsems = refs[10:13]
    ssems = refs[13:16]

    c = lax.axis_index("c")
    s = lax.axis_index("s")
    w = c * NS + s

    def idx_start(b, eoff):
        pltpu.async_copy(
            ei_hbm.at[pl.ds(0, 2), pl.ds(eoff, CH)], idx[b], isems[b])

    def idx_wait(b):
        pltpu.make_async_copy(
            ei_hbm.at[pl.ds(0, 2), pl.ds(0, CH)], idx[b], isems[b]).wait()

    def mask(b):
        for j in range(CH // LANES):
            sl = pl.ds(j * LANES, LANES)
            sv = idx[b][0, sl]
            dv = idx[b][1, sl]
            idx[b][1, sl] = jnp.where(
                sv == dv, jnp.full((LANES,), DUMMY, jnp.int32), dv)

    def g_start(b):
        pltpu.async_copy(feat_hbm.at[idx[b].at[0]], rows[b], gsems[b])

    def g_wait(b):
        pltpu.make_async_copy(feat_hbm.at[idx[b].at[0]], rows[b],
                              gsems[b]).wait()

    def s_start(b):
        pltpu.async_copy(rows[b], acc.at[idx[b].at[1]], ssems[b], add=True)

    def s_wait(b):
        pltpu.make_async_copy(rows[b], acc.at[idx[b].at[1]], ssems[b]).wait()

    @pl.loop(0, CH)
    def _(i):
        @pl.loop(0, D_C // LANES)
        def _(k):
            rows[0][i, pl.ds(k * LANES, LANES)] = jnp.zeros(
                (LANES,), jnp.float32)

    abase = s * ACC_PER_TILE

    @pl.loop(0, 4)
    def _(j):
        pltpu.sync_copy(rows[0], acc.at[pl.ds(abase + j * CH, CH)])
    pltpu.sync_copy(rows[0].at[pl.ds(0, ACC_PER_TILE - 4 * CH)],
                    acc.at[pl.ds(abase + 4 * CH, ACC_PER_TILE - 4 * CH)])

    plsc.subcore_barrier()

    ebase = w * (NCHT * CH)
    xoff = (NS * NC * NCHT + 2 * c + s) * CH

    def eoff_of(kk, static_k, n):
        if static_k == n - 1 and n == NCHT + 1:
            return xoff
        return ebase + kk * CH

    def do_iter(k, b, n, k_traced=None):
        kk = k if k_traced is None else k_traced
        b1, b2 = (b + 1) % 3, (b + 2) % 3
        if k + 1 < n:
            idx_wait(b1)
            mask(b1)
            g_start(b1)
        g_wait(b)
        s_start(b)
        if k >= 1:
            s_wait(b2)
        if k + 2 < n:
            idx_start(b2, eoff_of(kk + 2, k + 2, n))

    def pipeline(n):
        idx_start(0, eoff_of(0, 0, n))
        idx_start(1, eoff_of(1, 1, n))
        idx_wait(0)
        mask(0)
        g_start(0)

        nmain = ((n - 6) // 3) * 3

        for k in range(0, 3):
            do_iter(k, k % 3, n)

        @pl.loop(3, 3 + nmain, step=3)
        def _(k):
            do_iter(4, 0, n, k_traced=k)
            do_iter(4, 1, n, k_traced=k + 1)
            do_iter(4, 2, n, k_traced=k + 2)

        for k in range(3 + nmain, n):
            do_iter(k, k % 3, n)
        s_wait((n - 1) % 3)

    @pl.when(s < 2)
    def _():
        pipeline(NCHT + 1)

    @pl.when(s >= 2)
    def _():
        pipeline(NCHT)

    plsc.subcore_barrier()

    pltpu.sync_copy(acc.at[pl.ds(abase, ACC_PER_TILE)],
                    out_hbm.at[c].at[pl.ds(abase, ACC_PER_TILE)])


@jax.jit
def _sc_scatter(feat, edge_index):
    mesh = plsc.VectorSubcoreMesh(core_axis_name="c", subcore_axis_name="s")
    k = pl.kernel(
        _sc_body,
        out_type=jax.ShapeDtypeStruct((NC, ACC_ROWS, D_C), jnp.float32),
        mesh=mesh,
        scratch_types=(
            [pltpu.VMEM((2, CH), jnp.int32) for _ in range(3)]
            + [pltpu.VMEM((CH, D_C), jnp.float32) for _ in range(3)]
            + [pltpu.VMEM_SHARED((ACC_ROWS, D_C), jnp.float32)]
            + [pltpu.SemaphoreType.DMA for _ in range(9)]
        ),
    )
    return k(feat, edge_index)


def _finish_body(p0_ref, p1_ref, f_ref, w_ref, o_ref):
    x = p0_ref[0] + p1_ref[0] + f_ref[...]
    y = lax.dot_general(x, w_ref[...], (((1,), (0,)), ((), ())),
                        preferred_element_type=jnp.float32,
                        precision=lax.Precision.DEFAULT)
    o_ref[...] = jnp.maximum(y, 0.0)


BR = 2000


@jax.jit
def _tc_finish(partials, feat, weight1):
    return pl.pallas_call(
        _finish_body,
        grid=(N_NODES_C // BR,),
        in_specs=[
            pl.BlockSpec((1, BR, D_C), lambda i: (0, i, 0)),
            pl.BlockSpec((1, BR, D_C), lambda i: (1, i, 0)),
            pl.BlockSpec((BR, D_C), lambda i: (i, 0)),
            pl.BlockSpec((D_C, D_C), lambda i: (0, 0)),
        ],
        out_specs=pl.BlockSpec((BR, D_C), lambda i: (i, 0)),
        out_shape=jax.ShapeDtypeStruct((N_NODES_C, D_C), jnp.float32),
    )(partials, partials, feat, weight1)


def kernel(feat, edge_index, weight1):
    ei = edge_index.astype(jnp.int32)
    partials = _sc_scatter(feat, ei)
    return _tc_finish(partials, feat, weight1)
